# baseline (speedup 1.0000x reference)
import dataclasses
import functools

import jax
import jax.numpy as jnp
from jax import lax
from jax.experimental import pallas as pl
from jax.experimental.pallas import tpu as pltpu
from jax.experimental.pallas import tpu_sc as plsc

_N = 10000
_E = 320000
_D = 128
_H = 8
_F = 16
_T = 5

_NC = 2
_NS = 16
_NW = _NC * _NS
_EW = _E // _NW
_CH = 40
_NCHUNK = _EW // _CH
_DRB = 640
_RP = 40

_mesh = plsc.VectorSubcoreMesh(core_axis_name="c", subcore_axis_name="s")

_cp = pltpu.CompilerParams()
if "needs_layout_passes" in pltpu.CompilerParams.__dataclass_fields__:
    _cp = dataclasses.replace(_cp, needs_layout_passes=False)


@functools.partial(
    pl.kernel,
    mesh=_mesh,
    compiler_params=_cp,
    out_type=jax.ShapeDtypeStruct((_NC, _N, _D), jnp.float32),
    scratch_types=[
        pltpu.VMEM((2, _CH), jnp.int32),
        pltpu.VMEM((2, _CH), jnp.int32),
        pltpu.VMEM((2, _CH), jnp.int32),
        pltpu.VMEM((2, _CH, _D), jnp.float32),
        pltpu.VMEM((8, _D), jnp.float32),
        pltpu.VMEM((_T, 16), jnp.float32),
        pltpu.VMEM_SHARED((_N, _D), jnp.float32),
        pltpu.SemaphoreType.DMA((2,)),
        pltpu.SemaphoreType.DMA((2,)),
    ],
)
def _sc_denom(dst_hbm, ef_hbm, embp_hbm, out_hbm, tv2, dv2, dsc, ev2,
              embl, exq16, dacc, si, sc):
    c = lax.axis_index("c")
    s = lax.axis_index("s")
    np_s = jnp.where(s < _NS - 1, _DRB // _RP, 400 // _RP)
    z = jnp.zeros((16,), jnp.float32)

    pltpu.sync_copy(embp_hbm, embl)

    @pl.loop(0, _T)
    def _(t):
        exq16[t] = jnp.exp(embl[t, pl.ds(0, 16)])

    for b in range(2):
        @pl.loop(0, _CH)
        def _(i):
            for j in range(_D // 16):
                ev2[b, i, pl.ds(j * 16, 16)] = z

    @pl.loop(0, np_s)
    def _(p):
        pltpu.sync_copy(ev2.at[0], dacc.at[pl.ds(s * _DRB + p * _RP, _RP)])

    plsc.subcore_barrier()

    base = (c * _NS + s) * _EW

    def issue_idx(q, b):
        eb = base + q * _CH
        pltpu.async_copy(ef_hbm.at[pl.ds(eb, _CH)], tv2.at[b], si.at[b])
        pltpu.async_copy(dst_hbm.at[pl.ds(eb, _CH)], dv2.at[b], si.at[b])

    def wait_idx(b):
        pltpu.make_async_copy(
            ef_hbm.at[pl.ds(base, _CH)], tv2.at[b], si.at[b]).wait()
        pltpu.make_async_copy(
            dst_hbm.at[pl.ds(base, _CH)], dv2.at[b], si.at[b]).wait()

    def wait_scat(b):
        pltpu.make_async_copy(
            ev2.at[b], dacc.at[dsc.at[b]], sc.at[b]).wait()

    issue_idx(0, 0)
    issue_idx(1, 1)

    @pl.loop(0, _NCHUNK // 2)
    def _(k):
        for b in range(2):
            q = 2 * k + b
            wait_idx(b)

            @pl.when(k > 0)
            def _():
                wait_scat(b)

            for o in (0, 16, _CH - 16):
                dsc[b, pl.ds(o, 16)] = dv2[b, pl.ds(o, 16)]

            for o in (0, 16, _CH - 16):
                tvec = tv2[b, pl.ds(o, 16)]
                for j in range(16):
                    ev2[b, o + j, pl.ds(0, 16)] = exq16[tvec[j]]

            pltpu.async_copy(ev2.at[b], dacc.at[dsc.at[b]], sc.at[b],
                             add=True)

            @pl.when(q + 2 < _NCHUNK)
            def _():
                issue_idx(q + 2, b)

    wait_scat(0)
    wait_scat(1)
    plsc.subcore_barrier()

    @pl.loop(0, np_s)
    def _(p):
        r0 = s * _DRB + p * _RP
        pltpu.sync_copy(dacc.at[pl.ds(r0, _RP)], ev2.at[0])
        pltpu.sync_copy(ev2.at[0], out_hbm.at[c, pl.ds(r0, _RP)])


@functools.partial(
    pl.kernel,
    mesh=_mesh,
    compiler_params=_cp,
    out_type=(
        jax.ShapeDtypeStruct((_NC, _N, _D), jnp.float32),
        jax.ShapeDtypeStruct((_E, 16), jnp.float32),
    ),
    scratch_types=[
        pltpu.VMEM((2, 2, 2 * _CH), jnp.int32),
        pltpu.VMEM((2, _CH), jnp.int32),
        pltpu.VMEM((2, 2 * _CH, _D), jnp.float32),
        pltpu.VMEM((2, _CH, 16), jnp.float32),
        pltpu.VMEM_SHARED((_N, _D), jnp.float32),
        pltpu.SemaphoreType.DMA((2,)),
        pltpu.SemaphoreType.DMA((2,)),
        pltpu.SemaphoreType.DMA((2,)),
        pltpu.SemaphoreType.DMA((2,)),
    ],
)
def _sc_main(cidx_hbm, ctab_hbm, ft_hbm, attn_hbm,
             ci2, dsc, gbuf2, ab2, ftacc, si, sg, sc, sa):
    c = lax.axis_index("c")
    s = lax.axis_index("s")
    np_s = jnp.where(s < _NS - 1, _DRB // _RP, 400 // _RP)
    z = jnp.zeros((16,), jnp.float32)

    @pl.loop(0, _RP)
    def _(i):
        for j in range(_D // 16):
            gbuf2[0, i, pl.ds(j * 16, 16)] = z

    @pl.loop(0, np_s)
    def _(p):
        pltpu.sync_copy(gbuf2.at[0, pl.ds(0, _RP)],
                        ftacc.at[pl.ds(s * _DRB + p * _RP, _RP)])

    plsc.subcore_barrier()

    cbase = (c * _NS + s) * _NCHUNK

    def issue_idx(q, b):
        pltpu.async_copy(cidx_hbm.at[cbase + q], ci2.at[b], si.at[b])

    def wait_idx(b):
        pltpu.make_async_copy(cidx_hbm.at[cbase], ci2.at[b], si.at[b]).wait()

    def issue_gather(b):
        pltpu.async_copy(ctab_hbm.at[ci2.at[b].at[0]], gbuf2.at[b],
                         sg.at[b])

    def wait_gather(b):
        pltpu.make_async_copy(
            ctab_hbm.at[ci2.at[b].at[0]], gbuf2.at[b], sg.at[b]).wait()

    def wait_out(b):
        pltpu.make_async_copy(
            gbuf2.at[b, pl.ds(0, _CH)], ftacc.at[dsc.at[b]],
            sc.at[b]).wait()
        pltpu.make_async_copy(
            ab2.at[b], attn_hbm.at[pl.ds(0, _CH)], sa.at[b]).wait()

    issue_idx(0, 0)
    issue_idx(1, 1)
    wait_idx(0)
    issue_gather(0)

    @pl.loop(0, _NCHUNK // 2)
    def _(k):
        for b in range(2):
            q = 2 * k + b
            b1 = 1 - b

            if b == 0:
                wait_idx(b1)

                @pl.when(k > 0)
                def _():
                    wait_out(b1)

                issue_gather(b1)
            else:
                @pl.when(k + 1 < _NCHUNK // 2)
                def _():
                    wait_idx(b1)

                wait_out(b1)

                @pl.when(k + 1 < _NCHUNK // 2)
                def _():
                    issue_gather(b1)

            wait_gather(b)

            for o in (0, 16, _CH - 16):
                dsc[b, pl.ds(o, 16)] = ci2[b, 1, pl.ds(o, 16)]

            @pl.loop(0, _CH)
            def _(i):
                w16 = gbuf2[b, _CH + i, pl.ds(0, 16)]
                ab2[b, i] = w16
                b0 = jnp.full((16,), w16[0], jnp.float32)
                gbuf2[b, i, pl.ds(0, 16)] = gbuf2[b, i, pl.ds(0, 16)] * b0
                for h in range(1, _H):
                    hs = pl.ds(h * 16, 16)
                    gbuf2[b, i, hs] = gbuf2[b, i, hs] * gbuf2[b, _CH + i, hs]

            pltpu.async_copy(gbuf2.at[b, pl.ds(0, _CH)], ftacc.at[dsc.at[b]],
                             sc.at[b], add=True)
            eb = (cbase + q) * _CH
            pltpu.async_copy(ab2.at[b], attn_hbm.at[pl.ds(eb, _CH)],
                             sa.at[b])

            @pl.when(q + 2 < _NCHUNK)
            def _():
                issue_idx(q + 2, b)

    wait_out(1)
    plsc.subcore_barrier()

    @pl.loop(0, np_s)
    def _(p):
        r0 = s * _DRB + p * _RP
        pltpu.sync_copy(ftacc.at[pl.ds(r0, _RP)], gbuf2.at[0, pl.ds(0, _RP)])
        pltpu.sync_copy(gbuf2.at[0, pl.ds(0, _RP)], ft_hbm.at[c, pl.ds(r0, _RP)])


def _mm_body(f_ref, w_ref, o_ref):
    o_ref[...] = jnp.dot(f_ref[...], w_ref[...],
                         preferred_element_type=jnp.float32)


def _project(feat, wt):
    return pl.pallas_call(
        _mm_body,
        grid=(10,),
        in_specs=[
            pl.BlockSpec((_N // 10, _D), lambda i: (i, 0)),
            pl.BlockSpec((_D, _D), lambda i: (0, 0)),
        ],
        out_specs=pl.BlockSpec((_N // 10, _D), lambda i: (i, 0)),
        out_shape=jax.ShapeDtypeStruct((_N, _D), jnp.float32),
    )(feat, wt)


def _wt_body(dp_ref, emb_ref, o_ref):
    bn = _N // 10
    d8 = dp_ref[0][:, :_H] + dp_ref[1][:, :_H]
    safe = jnp.where(d8 > 0, d8, 1.0)
    rec = 1.0 / safe
    ex = jnp.exp(emb_ref[...])
    rows_t = []
    for t in range(_T):
        a = ex[t][None, :] * rec
        compact = jnp.concatenate(
            [a, jnp.zeros((bn, 8), jnp.float32)], axis=1)
        expanded = jnp.repeat(a[:, 1:], _F, axis=1)
        rows_t.append(jnp.concatenate([compact, expanded], axis=1))
    w = jnp.stack(rows_t, axis=1)
    o_ref[...] = w.reshape(bn * _T, _D)


def _weights(dparts, emb):
    return pl.pallas_call(
        _wt_body,
        grid=(10,),
        in_specs=[
            pl.BlockSpec((_NC, _N // 10, _D), lambda i: (0, i, 0)),
            pl.BlockSpec((_T, _H), lambda i: (0, 0)),
        ],
        out_specs=pl.BlockSpec((_N // 10 * _T, _D), lambda i: (i, 0)),
        out_shape=jax.ShapeDtypeStruct((_N * _T, _D), jnp.float32),
    )(dparts, emb)


def _add_body(p_ref, o_ref):
    o_ref[...] = p_ref[0] + p_ref[1]


def _combine(parts):
    return pl.pallas_call(
        _add_body,
        grid=(10,),
        in_specs=[pl.BlockSpec((_NC, _N // 10, _D), lambda i: (0, i, 0))],
        out_specs=pl.BlockSpec((_N // 10, _D), lambda i: (i, 0)),
        out_shape=jax.ShapeDtypeStruct((_N, _D), jnp.float32),
    )(parts)


def kernel(feat, edge_index, e_feat, W, edge_emb):
    src = edge_index[0].astype(jnp.int32)
    dst = edge_index[1].astype(jnp.int32)
    ef = e_feat.astype(jnp.int32)
    nt = dst * jnp.int32(_T) + ef
    embp = jnp.zeros((8, _D), jnp.float32).at[:_T, :_H].set(
        edge_emb.astype(jnp.float32))

    nch_all = _E // _CH
    srcr = src.reshape(nch_all, _CH)
    ntr = (nt + jnp.int32(_N)).reshape(nch_all, _CH)
    dstr = dst.reshape(nch_all, _CH)
    cidx = jnp.stack(
        [jnp.concatenate([srcr, ntr], axis=1),
         jnp.concatenate([dstr, jnp.zeros_like(dstr)], axis=1)], axis=1)

    fs = _project(feat, W.T)
    dparts = _sc_denom(dst, ef, embp)
    wexp = _weights(dparts, edge_emb)
    ctab = jnp.concatenate([fs, wexp], axis=0)
    ftparts, attn16 = _sc_main(cidx, ctab)
    rst = _combine(ftparts).reshape(_N, _H, _F)
    attn = attn16[:, :_H].reshape(_E, _H, 1)
    return rst, attn


# speedup vs baseline: 1.0533x; 1.0533x over previous
import dataclasses
import functools

import jax
import jax.numpy as jnp
from jax import lax
from jax.experimental import pallas as pl
from jax.experimental.pallas import tpu as pltpu
from jax.experimental.pallas import tpu_sc as plsc

_N = 10000
_E = 320000
_D = 128
_H = 8
_F = 16
_T = 5

_NC = 2
_NS = 16
_NW = _NC * _NS
_EW = _E // _NW
_CH = 40
_NCHUNK = _EW // _CH
_DRB = 640
_RP = 40

_mesh = plsc.VectorSubcoreMesh(core_axis_name="c", subcore_axis_name="s")

_cp = pltpu.CompilerParams()
if "needs_layout_passes" in pltpu.CompilerParams.__dataclass_fields__:
    _cp = dataclasses.replace(_cp, needs_layout_passes=False)


@functools.partial(
    pl.kernel,
    mesh=_mesh,
    compiler_params=_cp,
    out_type=jax.ShapeDtypeStruct((_NC, _N, _D), jnp.float32),
    scratch_types=[
        pltpu.VMEM((2, _CH), jnp.int32),
        pltpu.VMEM((2, _CH), jnp.int32),
        pltpu.VMEM((2, _CH), jnp.int32),
        pltpu.VMEM((2, _CH, _D), jnp.float32),
        pltpu.VMEM((8, _D), jnp.float32),
        pltpu.VMEM((_T, 16), jnp.float32),
        pltpu.VMEM_SHARED((_N, _D), jnp.float32),
        pltpu.SemaphoreType.DMA((2,)),
        pltpu.SemaphoreType.DMA((2,)),
    ],
)
def _sc_denom(dst_hbm, ef_hbm, embp_hbm, out_hbm, tv2, dv2, dsc, ev2,
              embl, exq16, dacc, si, sc):
    c = lax.axis_index("c")
    s = lax.axis_index("s")
    np_s = jnp.where(s < _NS - 1, _DRB // _RP, 400 // _RP)
    z = jnp.zeros((16,), jnp.float32)

    pltpu.sync_copy(embp_hbm, embl)

    @pl.loop(0, _T)
    def _(t):
        exq16[t] = jnp.exp(embl[t, pl.ds(0, 16)])

    for b in range(2):
        @pl.loop(0, _CH)
        def _(i):
            for j in range(_D // 16):
                ev2[b, i, pl.ds(j * 16, 16)] = z

    @pl.loop(0, np_s)
    def _(p):
        pltpu.sync_copy(ev2.at[0], dacc.at[pl.ds(s * _DRB + p * _RP, _RP)])

    plsc.subcore_barrier()

    base = (c * _NS + s) * _EW

    def issue_idx(q, b):
        eb = base + q * _CH
        pltpu.async_copy(ef_hbm.at[pl.ds(eb, _CH)], tv2.at[b], si.at[b])
        pltpu.async_copy(dst_hbm.at[pl.ds(eb, _CH)], dv2.at[b], si.at[b])

    def wait_idx(b):
        pltpu.make_async_copy(
            ef_hbm.at[pl.ds(base, _CH)], tv2.at[b], si.at[b]).wait()
        pltpu.make_async_copy(
            dst_hbm.at[pl.ds(base, _CH)], dv2.at[b], si.at[b]).wait()

    def wait_scat(b):
        pltpu.make_async_copy(
            ev2.at[b], dacc.at[dsc.at[b]], sc.at[b]).wait()

    issue_idx(0, 0)
    issue_idx(1, 1)

    @pl.loop(0, _NCHUNK // 2)
    def _(k):
        for b in range(2):
            q = 2 * k + b
            wait_idx(b)

            @pl.when(k > 0)
            def _():
                wait_scat(b)

            for o in (0, 16, _CH - 16):
                dsc[b, pl.ds(o, 16)] = dv2[b, pl.ds(o, 16)]

            for o in (0, 16, _CH - 16):
                tvec = tv2[b, pl.ds(o, 16)]
                for j in range(16):
                    ev2[b, o + j, pl.ds(0, 16)] = exq16[tvec[j]]

            pltpu.async_copy(ev2.at[b], dacc.at[dsc.at[b]], sc.at[b],
                             add=True)

            @pl.when(q + 2 < _NCHUNK)
            def _():
                issue_idx(q + 2, b)

    wait_scat(0)
    wait_scat(1)
    plsc.subcore_barrier()

    @pl.loop(0, np_s)
    def _(p):
        r0 = s * _DRB + p * _RP
        pltpu.async_copy(dacc.at[pl.ds(r0, _RP)],
                         out_hbm.at[c, pl.ds(r0, _RP)], si.at[0])

    @pl.loop(0, np_s)
    def _(p):
        pltpu.make_async_copy(
            dacc.at[pl.ds(0, _RP)], out_hbm.at[c, pl.ds(0, _RP)],
            si.at[0]).wait()


@functools.partial(
    pl.kernel,
    mesh=_mesh,
    compiler_params=_cp,
    out_type=(
        jax.ShapeDtypeStruct((_NC, _N, _D), jnp.float32),
        jax.ShapeDtypeStruct((_E, 16), jnp.float32),
    ),
    scratch_types=[
        pltpu.VMEM((2, _CH), jnp.int32),
        pltpu.VMEM((2, _CH), jnp.int32),
        pltpu.VMEM((2, _CH), jnp.int32),
        pltpu.VMEM((2, _CH), jnp.int32),
        pltpu.VMEM((2, _CH, _D), jnp.float32),
        pltpu.VMEM((2, _CH, _D), jnp.float32),
        pltpu.VMEM((2, _CH, 16), jnp.float32),
        pltpu.VMEM_SHARED((_N, _D), jnp.float32),
        pltpu.SemaphoreType.DMA((2,)),
        pltpu.SemaphoreType.DMA((2,)),
        pltpu.SemaphoreType.DMA((2,)),
        pltpu.SemaphoreType.DMA((2,)),
    ],
)
def _sc_main(src_hbm, nt_hbm, dst_hbm, fs_hbm, wexp_hbm, ft_hbm, attn_hbm,
             sv2, nv2, dv2, dsc, rows2, wbuf2, ab2, ftacc, si, sg, sc, sa):
    c = lax.axis_index("c")
    s = lax.axis_index("s")
    np_s = jnp.where(s < _NS - 1, _DRB // _RP, 400 // _RP)
    z = jnp.zeros((16,), jnp.float32)

    @pl.loop(0, _RP)
    def _(i):
        for j in range(_D // 16):
            rows2[0, i, pl.ds(j * 16, 16)] = z

    @pl.loop(0, np_s)
    def _(p):
        pltpu.sync_copy(rows2.at[0],
                        ftacc.at[pl.ds(s * _DRB + p * _RP, _RP)])

    plsc.subcore_barrier()

    base = (c * _NS + s) * _EW

    def issue_idx(q, b):
        eb = base + q * _CH
        pltpu.async_copy(src_hbm.at[pl.ds(eb, _CH)], sv2.at[b], si.at[b])
        pltpu.async_copy(nt_hbm.at[pl.ds(eb, _CH)], nv2.at[b], si.at[b])
        pltpu.async_copy(dst_hbm.at[pl.ds(eb, _CH)], dv2.at[b], si.at[b])

    def wait_idx(b):
        for ref in (sv2, nv2, dv2):
            pltpu.make_async_copy(
                src_hbm.at[pl.ds(base, _CH)], ref.at[b], si.at[b]).wait()

    def issue_gather(b):
        pltpu.async_copy(fs_hbm.at[sv2.at[b]], rows2.at[b], sg.at[b])
        pltpu.async_copy(wexp_hbm.at[nv2.at[b]], wbuf2.at[b], sg.at[b])

    def wait_gather(b):
        pltpu.make_async_copy(
            fs_hbm.at[sv2.at[b]], rows2.at[b], sg.at[b]).wait()
        pltpu.make_async_copy(
            wexp_hbm.at[nv2.at[b]], wbuf2.at[b], sg.at[b]).wait()

    def wait_out(b):
        pltpu.make_async_copy(
            rows2.at[b], ftacc.at[dsc.at[b]], sc.at[b]).wait()
        pltpu.make_async_copy(
            ab2.at[b], attn_hbm.at[pl.ds(base, _CH)], sa.at[b]).wait()

    issue_idx(0, 0)
    issue_idx(1, 1)
    wait_idx(0)
    issue_gather(0)

    @pl.loop(0, _NCHUNK // 2)
    def _(k):
        for b in range(2):
            q = 2 * k + b
            b1 = 1 - b

            if b == 0:
                wait_idx(b1)

                @pl.when(k > 0)
                def _():
                    wait_out(b1)

                issue_gather(b1)
            else:
                @pl.when(k + 1 < _NCHUNK // 2)
                def _():
                    wait_idx(b1)

                wait_out(b1)

                @pl.when(k + 1 < _NCHUNK // 2)
                def _():
                    issue_gather(b1)

            wait_gather(b)

            for o in (0, 16, _CH - 16):
                dsc[b, pl.ds(o, 16)] = dv2[b, pl.ds(o, 16)]

            @pl.loop(0, _CH)
            def _(i):
                w16 = wbuf2[b, i, pl.ds(0, 16)]
                ab2[b, i] = w16
                b0 = jnp.full((16,), w16[0], jnp.float32)
                rows2[b, i, pl.ds(0, 16)] = rows2[b, i, pl.ds(0, 16)] * b0
                for h in range(1, _H):
                    hs = pl.ds(h * 16, 16)
                    rows2[b, i, hs] = rows2[b, i, hs] * wbuf2[b, i, hs]

            pltpu.async_copy(rows2.at[b], ftacc.at[dsc.at[b]], sc.at[b],
                             add=True)
            eb = base + q * _CH
            pltpu.async_copy(ab2.at[b], attn_hbm.at[pl.ds(eb, _CH)],
                             sa.at[b])

            @pl.when(q + 2 < _NCHUNK)
            def _():
                issue_idx(q + 2, b)

    wait_out(1)
    plsc.subcore_barrier()

    @pl.loop(0, np_s)
    def _(p):
        r0 = s * _DRB + p * _RP
        pltpu.async_copy(ftacc.at[pl.ds(r0, _RP)],
                         ft_hbm.at[c, pl.ds(r0, _RP)], si.at[0])

    @pl.loop(0, np_s)
    def _(p):
        pltpu.make_async_copy(
            ftacc.at[pl.ds(0, _RP)], ft_hbm.at[c, pl.ds(0, _RP)],
            si.at[0]).wait()


def _mm_body(f_ref, w_ref, o_ref):
    o_ref[...] = jnp.dot(f_ref[...], w_ref[...],
                         preferred_element_type=jnp.float32)


def _project(feat, wt):
    return pl.pallas_call(
        _mm_body,
        grid=(10,),
        in_specs=[
            pl.BlockSpec((_N // 10, _D), lambda i: (i, 0)),
            pl.BlockSpec((_D, _D), lambda i: (0, 0)),
        ],
        out_specs=pl.BlockSpec((_N // 10, _D), lambda i: (i, 0)),
        out_shape=jax.ShapeDtypeStruct((_N, _D), jnp.float32),
    )(feat, wt)


def _wt_body(dp_ref, emb_ref, o_ref):
    bn = _N // 10
    d8 = dp_ref[0][:, :_H] + dp_ref[1][:, :_H]
    safe = jnp.where(d8 > 0, d8, 1.0)
    rec = 1.0 / safe
    ex = jnp.exp(emb_ref[...])
    rows_t = []
    for t in range(_T):
        a = ex[t][None, :] * rec
        compact = jnp.concatenate(
            [a, jnp.zeros((bn, 8), jnp.float32)], axis=1)
        expanded = jnp.repeat(a[:, 1:], _F, axis=1)
        rows_t.append(jnp.concatenate([compact, expanded], axis=1))
    w = jnp.stack(rows_t, axis=1)
    o_ref[...] = w.reshape(bn * _T, _D)


def _weights(dparts, emb):
    return pl.pallas_call(
        _wt_body,
        grid=(10,),
        in_specs=[
            pl.BlockSpec((_NC, _N // 10, _D), lambda i: (0, i, 0)),
            pl.BlockSpec((_T, _H), lambda i: (0, 0)),
        ],
        out_specs=pl.BlockSpec((_N // 10 * _T, _D), lambda i: (i, 0)),
        out_shape=jax.ShapeDtypeStruct((_N * _T, _D), jnp.float32),
    )(dparts, emb)


def _add_body(p_ref, o_ref):
    o_ref[...] = p_ref[0] + p_ref[1]


def _combine(parts):
    return pl.pallas_call(
        _add_body,
        grid=(10,),
        in_specs=[pl.BlockSpec((_NC, _N // 10, _D), lambda i: (0, i, 0))],
        out_specs=pl.BlockSpec((_N // 10, _D), lambda i: (i, 0)),
        out_shape=jax.ShapeDtypeStruct((_N, _D), jnp.float32),
    )(parts)


def kernel(feat, edge_index, e_feat, W, edge_emb):
    src = edge_index[0].astype(jnp.int32)
    dst = edge_index[1].astype(jnp.int32)
    ef = e_feat.astype(jnp.int32)
    nt = dst * jnp.int32(_T) + ef
    embp = jnp.zeros((8, _D), jnp.float32).at[:_T, :_H].set(
        edge_emb.astype(jnp.float32))

    fs = _project(feat, W.T)
    dparts = _sc_denom(dst, ef, embp)
    wexp = _weights(dparts, edge_emb)
    ftparts, attn16 = _sc_main(src, nt, dst, fs, wexp)
    rst = _combine(ftparts).reshape(_N, _H, _F)
    attn = attn16[:, :_H].reshape(_E, _H, 1)
    return rst, attn
